# baseline (speedup 1.0000x reference)
import functools

import jax
import jax.numpy as jnp
from jax import lax
from jax.experimental import pallas as pl
from jax.experimental.pallas import tpu as pltpu
from jax.experimental.pallas import tpu_sc as plsc

E = 8
K = 2
H = 2048
F = 768
T = 2048
S = T * K
TM = 256
NBLK = S // TM + E
NPAD = NBLK * TM

NC, NS = 2, 16
NW = NC * NS

def _sc_mesh():
    return plsc.VectorSubcoreMesh(core_axis_name="c", subcore_axis_name="s")


def _router_body(x_ref, gw_ref, w_ref, e_ref):
    x = x_ref[...]
    logits = lax.dot_general(x, gw_ref[...], (((1,), (1,)), ((), ())),
                             preferred_element_type=jnp.float32)
    m = jnp.max(logits, axis=1, keepdims=True)
    p = jnp.exp(logits - m)
    sm = p / jnp.sum(p, axis=1, keepdims=True)
    iota = lax.broadcasted_iota(jnp.int32, sm.shape, 1)
    m0 = jnp.max(sm, axis=1, keepdims=True)
    i0 = jnp.min(jnp.where(sm == m0, iota, E), axis=1, keepdims=True)
    masked = jnp.where(iota == i0, -1.0, sm)
    m1 = jnp.max(masked, axis=1, keepdims=True)
    i1 = jnp.min(jnp.where(masked == m1, iota, E), axis=1, keepdims=True)
    ssum = m0 + m1
    w_ref[...] = jnp.concatenate([m0 / ssum, m1 / ssum], axis=1)
    e_ref[...] = jnp.concatenate([i0, i1], axis=1)


def _router(x, gate_w):
    return pl.pallas_call(
        _router_body,
        out_shape=[jax.ShapeDtypeStruct((T, K), jnp.float32),
                   jax.ShapeDtypeStruct((T, K), jnp.int32)],
    )(x, gate_w)


def _dispatch_plan(e2, w2):
    e_flat = e2.reshape(-1)
    w_flat = w2.reshape(-1)
    oh = (e_flat[:, None] == jnp.arange(E, dtype=jnp.int32)[None, :])
    cum = jnp.cumsum(oh.astype(jnp.int32), axis=0)
    rank = jnp.take_along_axis(cum, e_flat[:, None], axis=1)[:, 0] - 1
    counts = cum[-1]
    nblk_e = (counts + TM - 1) // TM
    blk_cum = jnp.cumsum(nblk_e)
    n_active = blk_cum[-1]
    blk_start = blk_cum - nblk_e
    pos = blk_start[e_flat] * TM + rank
    tok = jnp.arange(S, dtype=jnp.int32) // K
    row_ids = jnp.zeros((NPAD,), jnp.int32).at[pos].set(tok)
    wrow = jnp.zeros((NPAD,), jnp.float32).at[pos].set(w_flat)
    bids = jnp.arange(NBLK, dtype=jnp.int32)
    bexp = jnp.searchsorted(blk_cum, bids, side="right").astype(jnp.int32)
    last_e = jnp.searchsorted(blk_cum, n_active - 1, side="right").astype(jnp.int32)
    bexp = jnp.where(bids < n_active, bexp, last_e)
    bxs = jnp.where(bids < n_active, bids, n_active - 1).astype(jnp.int32)
    nact = jnp.reshape(n_active, (1,)).astype(jnp.int32)
    p01 = pos.reshape(T, K).astype(jnp.int32)
    return row_ids, wrow, bexp, bxs, nact, p01


GCH = 16
G_PER_W = NPAD // NW


def _gather_sc_body(x_hbm, ids_hbm, out_hbm, idx_v, rows_v, sem):
    wid = lax.axis_index("s") * NC + lax.axis_index("c")
    base = wid * G_PER_W
    for c in range(G_PER_W // GCH):
        o = base + c * GCH
        pltpu.sync_copy(ids_hbm.at[pl.ds(o, GCH)], idx_v)
        pltpu.async_copy(x_hbm.at[idx_v], rows_v, sem).wait()
        pltpu.sync_copy(rows_v, out_hbm.at[pl.ds(o, GCH)])


def _gather_rows(x, ids):
    return pl.kernel(
        _gather_sc_body,
        out_type=jax.ShapeDtypeStruct((NPAD, H), jnp.float32),
        mesh=_sc_mesh(),
        scratch_types=[
            pltpu.VMEM((GCH,), jnp.int32),
            pltpu.VMEM((GCH, H), jnp.float32),
            pltpu.SemaphoreType.DMA,
        ],
    )(x, ids)


def _mlp_body(be_ref, bx_ref, na_ref, xs_ref, wr_ref, wg_ref, wu_ref, wd_ref,
              out_ref):
    b = pl.program_id(0)

    @pl.when(b < na_ref[0])
    def _():
        xb = xs_ref[...]
        h = jnp.dot(xb, wg_ref[0], preferred_element_type=jnp.float32)
        u = jnp.dot(xb, wu_ref[0], preferred_element_type=jnp.float32)
        glu = (h * (1.0 / (1.0 + jnp.exp(-h)))) * u
        glu = glu * wr_ref[:, 0:1]
        out_ref[...] = jnp.dot(glu, wd_ref[0], preferred_element_type=jnp.float32)


def _mlp(xs, wrep, w_gate, w_up, w_down, bexp, bxs, nact):
    grid_spec = pltpu.PrefetchScalarGridSpec(
        num_scalar_prefetch=3,
        grid=(NBLK,),
        in_specs=[
            pl.BlockSpec((TM, H), lambda b, be, bx, na: (bx[b], 0)),
            pl.BlockSpec((TM, 128), lambda b, be, bx, na: (bx[b], 0)),
            pl.BlockSpec((1, H, F), lambda b, be, bx, na: (be[b], 0, 0)),
            pl.BlockSpec((1, H, F), lambda b, be, bx, na: (be[b], 0, 0)),
            pl.BlockSpec((1, F, H), lambda b, be, bx, na: (be[b], 0, 0)),
        ],
        out_specs=pl.BlockSpec((TM, H), lambda b, be, bx, na: (bx[b], 0)),
    )
    return pl.pallas_call(
        _mlp_body,
        grid_spec=grid_spec,
        out_shape=jax.ShapeDtypeStruct((NPAD, H), jnp.float32),
        compiler_params=pltpu.CompilerParams(
            dimension_semantics=("arbitrary",)),
    )(bexp, bxs, nact, xs, wrep, w_gate, w_up, w_down)


CCH = 8
C_PER_W = T // NW


def _combine_sc_body(rows_hbm, p0_hbm, p1_hbm, out_hbm,
                     idx0_v, idx1_v, a_v, b_v, sem):
    wid = lax.axis_index("s") * NC + lax.axis_index("c")
    tbase = wid * C_PER_W
    for c in range(C_PER_W // CCH):
        o = tbase + c * CCH
        pltpu.sync_copy(p0_hbm.at[pl.ds(o, CCH)], idx0_v)
        pltpu.sync_copy(p1_hbm.at[pl.ds(o, CCH)], idx1_v)
        pltpu.async_copy(rows_hbm.at[idx0_v], a_v, sem).wait()
        pltpu.async_copy(rows_hbm.at[idx1_v], b_v, sem).wait()

        def row_body(r, _):
            def col_body(jj, _):
                for u in range(4):
                    off = jj * 64 + u * 16
                    a_v[r, pl.ds(off, 16)] = (a_v[r, pl.ds(off, 16)] +
                                              b_v[r, pl.ds(off, 16)])
                return 0
            return lax.fori_loop(0, H // 64, col_body, 0)

        lax.fori_loop(0, CCH, row_body, 0)
        pltpu.sync_copy(a_v, out_hbm.at[pl.ds(o, CCH)])


def _combine(yrows, p0, p1):
    return pl.kernel(
        _combine_sc_body,
        out_type=jax.ShapeDtypeStruct((T, H), jnp.float32),
        mesh=_sc_mesh(),
        scratch_types=[
            pltpu.VMEM((CCH,), jnp.int32),
            pltpu.VMEM((CCH,), jnp.int32),
            pltpu.VMEM((CCH, H), jnp.float32),
            pltpu.VMEM((CCH, H), jnp.float32),
            pltpu.SemaphoreType.DMA,
        ],
    )(yrows, p0, p1)


def kernel(hidden_states, gate_w, w_gate, w_up, w_down):
    Bb, Ss, Dd = hidden_states.shape
    x = hidden_states.reshape(-1, Dd)
    w2, e2 = _router(x, gate_w)
    row_ids, wrow, bexp, bxs, nact, p01 = _dispatch_plan(e2, w2)
    wrep = jnp.broadcast_to(wrow[:, None], (NPAD, 128))
    xs = _gather_rows(x, row_ids)
    yrows = _mlp(xs, wrep, w_gate, w_up, w_down, bexp, bxs, nact)
    y = _combine(yrows, p01[:, 0], p01[:, 1])
    return y.reshape(Bb, Ss, Dd)


# speedup vs baseline: 1.8333x; 1.8333x over previous
import functools

import jax
import jax.numpy as jnp
from jax import lax
from jax.experimental import pallas as pl
from jax.experimental.pallas import tpu as pltpu
from jax.experimental.pallas import tpu_sc as plsc

E = 8
K = 2
H = 2048
F = 768
T = 2048
S = T * K
TM = 256
NBLK = S // TM + E
NPAD = NBLK * TM

NC, NS = 2, 16
NW = NC * NS

def _sc_mesh():
    return plsc.VectorSubcoreMesh(core_axis_name="c", subcore_axis_name="s")


def _router_body(x_ref, gw_ref, w_ref, e_ref):
    x = x_ref[...]
    logits = lax.dot_general(x, gw_ref[...], (((1,), (1,)), ((), ())),
                             preferred_element_type=jnp.float32)
    m = jnp.max(logits, axis=1, keepdims=True)
    p = jnp.exp(logits - m)
    sm = p / jnp.sum(p, axis=1, keepdims=True)
    iota = lax.broadcasted_iota(jnp.int32, sm.shape, 1)
    m0 = jnp.max(sm, axis=1, keepdims=True)
    i0 = jnp.min(jnp.where(sm == m0, iota, E), axis=1, keepdims=True)
    masked = jnp.where(iota == i0, -1.0, sm)
    m1 = jnp.max(masked, axis=1, keepdims=True)
    i1 = jnp.min(jnp.where(masked == m1, iota, E), axis=1, keepdims=True)
    ssum = m0 + m1
    w_ref[...] = jnp.concatenate([m0 / ssum, m1 / ssum], axis=1)
    e_ref[...] = jnp.concatenate([i0, i1], axis=1)


def _router(x, gate_w):
    return pl.pallas_call(
        _router_body,
        out_shape=[jax.ShapeDtypeStruct((T, K), jnp.float32),
                   jax.ShapeDtypeStruct((T, K), jnp.int32)],
    )(x, gate_w)


def _dispatch_plan(e2, w2):
    e_flat = e2.reshape(-1)
    w_flat = w2.reshape(-1)
    oh = (e_flat[:, None] == jnp.arange(E, dtype=jnp.int32)[None, :])
    cum = jnp.cumsum(oh.astype(jnp.int32), axis=0)
    rank = jnp.take_along_axis(cum, e_flat[:, None], axis=1)[:, 0] - 1
    counts = cum[-1]
    nblk_e = (counts + TM - 1) // TM
    blk_cum = jnp.cumsum(nblk_e)
    n_active = blk_cum[-1]
    blk_start = blk_cum - nblk_e
    pos = blk_start[e_flat] * TM + rank
    tok = jnp.arange(S, dtype=jnp.int32) // K
    row_ids = jnp.zeros((NPAD,), jnp.int32).at[pos].set(tok)
    wrow = jnp.zeros((NPAD,), jnp.float32).at[pos].set(w_flat)
    bids = jnp.arange(NBLK, dtype=jnp.int32)
    bexp = jnp.searchsorted(blk_cum, bids, side="right").astype(jnp.int32)
    last_e = jnp.searchsorted(blk_cum, n_active - 1, side="right").astype(jnp.int32)
    bexp = jnp.where(bids < n_active, bexp, last_e)
    bxs = jnp.where(bids < n_active, bids, n_active - 1).astype(jnp.int32)
    nact = jnp.reshape(n_active, (1,)).astype(jnp.int32)
    p01 = pos.reshape(T, K).astype(jnp.int32)
    return row_ids, wrow, bexp, bxs, nact, p01


SCH = 8
C_PER_W = T // NW
SNCH = C_PER_W // SCH


def _scatter_sc_body(x_hbm, p0_hbm, p1_hbm, out_hbm,
                     idx0_v, idx1_v, rows_v0, rows_v1,
                     sin0, sin1, sout0, sout1):
    wid = lax.axis_index("s") * NC + lax.axis_index("c")
    tbase = wid * C_PER_W
    pltpu.sync_copy(p0_hbm.at[wid], idx0_v)
    pltpu.sync_copy(p1_hbm.at[wid], idx1_v)
    bufs = (rows_v0, rows_v1)
    sins = (sin0, sin1)
    souts = (sout0, sout1)
    gin = [None, None]
    gout = [[], []]
    gin[0] = pltpu.async_copy(x_hbm.at[pl.ds(tbase, SCH)], bufs[0], sins[0])
    for c in range(SNCH):
        b = c & 1
        if c + 1 < SNCH:
            b2 = (c + 1) & 1
            for cp in gout[b2]:
                cp.wait()
            gout[b2] = []
            gin[b2] = pltpu.async_copy(
                x_hbm.at[pl.ds(tbase + (c + 1) * SCH, SCH)], bufs[b2], sins[b2])
        gin[b].wait()
        gout[b] = [
            pltpu.async_copy(bufs[b], out_hbm.at[idx0_v.at[c]], souts[b]),
            pltpu.async_copy(bufs[b], out_hbm.at[idx1_v.at[c]], souts[b]),
        ]
    for b in (0, 1):
        for cp in gout[b]:
            cp.wait()


def _scatter_rows(x, p0w, p1w):
    return pl.kernel(
        _scatter_sc_body,
        out_type=jax.ShapeDtypeStruct((NPAD, H), jnp.float32),
        mesh=_sc_mesh(),
        scratch_types=[
            pltpu.VMEM((SNCH, SCH), jnp.int32),
            pltpu.VMEM((SNCH, SCH), jnp.int32),
            pltpu.VMEM((SCH, H), jnp.float32),
            pltpu.VMEM((SCH, H), jnp.float32),
            pltpu.SemaphoreType.DMA,
            pltpu.SemaphoreType.DMA,
            pltpu.SemaphoreType.DMA,
            pltpu.SemaphoreType.DMA,
        ],
    )(x, p0w, p1w)


def _mlp_body(be_ref, bx_ref, na_ref, xs_ref, wr_ref, wg_ref, wu_ref, wd_ref,
              out_ref):
    b = pl.program_id(0)

    @pl.when(b < na_ref[0])
    def _():
        xb = xs_ref[...]
        h = jnp.dot(xb, wg_ref[0], preferred_element_type=jnp.float32)
        u = jnp.dot(xb, wu_ref[0], preferred_element_type=jnp.float32)
        glu = (h * (1.0 / (1.0 + jnp.exp(-h)))) * u
        glu = glu * wr_ref[:, 0:1]
        out_ref[...] = jnp.dot(glu, wd_ref[0], preferred_element_type=jnp.float32)


def _mlp(xs, wrep, w_gate, w_up, w_down, bexp, bxs, nact):
    grid_spec = pltpu.PrefetchScalarGridSpec(
        num_scalar_prefetch=3,
        grid=(NBLK,),
        in_specs=[
            pl.BlockSpec((TM, H), lambda b, be, bx, na: (bx[b], 0)),
            pl.BlockSpec((TM, 128), lambda b, be, bx, na: (bx[b], 0)),
            pl.BlockSpec((1, H, F), lambda b, be, bx, na: (be[b], 0, 0)),
            pl.BlockSpec((1, H, F), lambda b, be, bx, na: (be[b], 0, 0)),
            pl.BlockSpec((1, F, H), lambda b, be, bx, na: (be[b], 0, 0)),
        ],
        out_specs=pl.BlockSpec((TM, H), lambda b, be, bx, na: (bx[b], 0)),
    )
    return pl.pallas_call(
        _mlp_body,
        grid_spec=grid_spec,
        out_shape=jax.ShapeDtypeStruct((NPAD, H), jnp.float32),
        compiler_params=pltpu.CompilerParams(
            dimension_semantics=("arbitrary",)),
    )(bexp, bxs, nact, xs, wrep, w_gate, w_up, w_down)


CCH = 8
CNCH = C_PER_W // CCH


def _combine_sc_body(rows_hbm, p0_hbm, p1_hbm, out_hbm,
                     idx0_v, idx1_v, a_v0, b_v0, a_v1, b_v1,
                     sg0, sg1, so0, so1):
    wid = lax.axis_index("s") * NC + lax.axis_index("c")
    tbase = wid * C_PER_W
    pltpu.sync_copy(p0_hbm.at[pl.ds(tbase, C_PER_W)], idx0_v)
    pltpu.sync_copy(p1_hbm.at[pl.ds(tbase, C_PER_W)], idx1_v)
    abufs = (a_v0, a_v1)
    bbufs = (b_v0, b_v1)
    sgs = (sg0, sg1)
    sos = (so0, so1)

    def fire(c, s):
        return [
            pltpu.async_copy(rows_hbm.at[idx0_v.at[pl.ds(c * CCH, CCH)]],
                             abufs[s], sgs[s]),
            pltpu.async_copy(rows_hbm.at[idx1_v.at[pl.ds(c * CCH, CCH)]],
                             bbufs[s], sgs[s]),
        ]

    gcp = [[], []]
    ocp = [None, None]
    gcp[0] = fire(0, 0)
    for c in range(CNCH):
        s = c & 1
        if c + 1 < CNCH:
            s2 = (c + 1) & 1
            if ocp[s2] is not None:
                ocp[s2].wait()
                ocp[s2] = None
            gcp[s2] = fire(c + 1, s2)
        for cp in gcp[s]:
            cp.wait()
        a_v, b_v = abufs[s], bbufs[s]

        def row_body(r, _):
            def col_body(jj, _):
                for u in range(8):
                    off = jj * 128 + u * 16
                    a_v[r, pl.ds(off, 16)] = (a_v[r, pl.ds(off, 16)] +
                                              b_v[r, pl.ds(off, 16)])
                return 0
            return lax.fori_loop(0, H // 128, col_body, 0)

        lax.fori_loop(0, CCH, row_body, 0)
        ocp[s] = pltpu.async_copy(
            a_v, out_hbm.at[pl.ds(tbase + c * CCH, CCH)], sos[s])
    for s in (0, 1):
        if ocp[s] is not None:
            ocp[s].wait()


def _combine(yrows, p0, p1):
    return pl.kernel(
        _combine_sc_body,
        out_type=jax.ShapeDtypeStruct((T, H), jnp.float32),
        mesh=_sc_mesh(),
        scratch_types=[
            pltpu.VMEM((C_PER_W,), jnp.int32),
            pltpu.VMEM((C_PER_W,), jnp.int32),
            pltpu.VMEM((CCH, H), jnp.float32),
            pltpu.VMEM((CCH, H), jnp.float32),
            pltpu.VMEM((CCH, H), jnp.float32),
            pltpu.VMEM((CCH, H), jnp.float32),
            pltpu.SemaphoreType.DMA,
            pltpu.SemaphoreType.DMA,
            pltpu.SemaphoreType.DMA,
            pltpu.SemaphoreType.DMA,
        ],
    )(yrows, p0, p1)


def kernel(hidden_states, gate_w, w_gate, w_up, w_down):
    Bb, Ss, Dd = hidden_states.shape
    x = hidden_states.reshape(-1, Dd)
    w2, e2 = _router(x, gate_w)
    row_ids, wrow, bexp, bxs, nact, p01 = _dispatch_plan(e2, w2)
    wrep = jnp.broadcast_to(wrow[:, None], (NPAD, 128))
    p0w = p01[:, 0].reshape(NW, SNCH, SCH)
    p1w = p01[:, 1].reshape(NW, SNCH, SCH)
    xs = _scatter_rows(x, p0w, p1w)
    yrows = _mlp(xs, wrep, w_gate, w_up, w_down, bexp, bxs, nact)
    y = _combine(yrows, p01[:, 0], p01[:, 1])
    return y.reshape(Bb, Ss, Dd)


# speedup vs baseline: 1.8416x; 1.0045x over previous
import functools

import jax
import jax.numpy as jnp
from jax import lax
from jax.experimental import pallas as pl
from jax.experimental.pallas import tpu as pltpu
from jax.experimental.pallas import tpu_sc as plsc

E = 8
K = 2
H = 2048
F = 768
T = 2048
S = T * K
TM = 512
NBLK = S // TM + E
NPAD = NBLK * TM

NC, NS = 2, 16
NW = NC * NS

def _sc_mesh():
    return plsc.VectorSubcoreMesh(core_axis_name="c", subcore_axis_name="s")


def _router_body(x_ref, gw_ref, w_ref, e_ref):
    x = x_ref[...]
    logits = lax.dot_general(x, gw_ref[...], (((1,), (1,)), ((), ())),
                             preferred_element_type=jnp.float32)
    m = jnp.max(logits, axis=1, keepdims=True)
    p = jnp.exp(logits - m)
    sm = p / jnp.sum(p, axis=1, keepdims=True)
    iota = lax.broadcasted_iota(jnp.int32, sm.shape, 1)
    m0 = jnp.max(sm, axis=1, keepdims=True)
    i0 = jnp.min(jnp.where(sm == m0, iota, E), axis=1, keepdims=True)
    masked = jnp.where(iota == i0, -1.0, sm)
    m1 = jnp.max(masked, axis=1, keepdims=True)
    i1 = jnp.min(jnp.where(masked == m1, iota, E), axis=1, keepdims=True)
    ssum = m0 + m1
    w_ref[...] = jnp.concatenate([m0 / ssum, m1 / ssum], axis=1)
    e_ref[...] = jnp.concatenate([i0, i1], axis=1)


def _router(x, gate_w):
    return pl.pallas_call(
        _router_body,
        out_shape=[jax.ShapeDtypeStruct((T, K), jnp.float32),
                   jax.ShapeDtypeStruct((T, K), jnp.int32)],
    )(x, gate_w)


def _dispatch_plan(e2, w2):
    e_flat = e2.reshape(-1)
    w_flat = w2.reshape(-1)
    oh = (e_flat[:, None] == jnp.arange(E, dtype=jnp.int32)[None, :])
    cum = jnp.cumsum(oh.astype(jnp.int32), axis=0)
    rank = jnp.take_along_axis(cum, e_flat[:, None], axis=1)[:, 0] - 1
    counts = cum[-1]
    nblk_e = (counts + TM - 1) // TM
    blk_cum = jnp.cumsum(nblk_e)
    n_active = blk_cum[-1]
    blk_start = blk_cum - nblk_e
    pos = blk_start[e_flat] * TM + rank
    tok = jnp.arange(S, dtype=jnp.int32) // K
    row_ids = jnp.zeros((NPAD,), jnp.int32).at[pos].set(tok)
    wrow = jnp.zeros((NPAD,), jnp.float32).at[pos].set(w_flat)
    bids = jnp.arange(NBLK, dtype=jnp.int32)
    bexp = jnp.searchsorted(blk_cum, bids, side="right").astype(jnp.int32)
    last_e = jnp.searchsorted(blk_cum, n_active - 1, side="right").astype(jnp.int32)
    bexp = jnp.where(bids < n_active, bexp, last_e)
    bxs = jnp.where(bids < n_active, bids, n_active - 1).astype(jnp.int32)
    nact = jnp.reshape(n_active, (1,)).astype(jnp.int32)
    p01 = pos.reshape(T, K).astype(jnp.int32)
    return row_ids, wrow, bexp, bxs, nact, p01


SCH = 8
C_PER_W = T // NW
SNCH = C_PER_W // SCH


def _scatter_sc_body(x_hbm, p0_hbm, p1_hbm, out_hbm,
                     idx0_v, idx1_v, rows_v0, rows_v1,
                     sin0, sin1, sout0, sout1):
    wid = lax.axis_index("s") * NC + lax.axis_index("c")
    tbase = wid * C_PER_W
    pltpu.sync_copy(p0_hbm.at[wid], idx0_v)
    pltpu.sync_copy(p1_hbm.at[wid], idx1_v)
    bufs = (rows_v0, rows_v1)
    sins = (sin0, sin1)
    souts = (sout0, sout1)
    gin = [None, None]
    gout = [[], []]
    gin[0] = pltpu.async_copy(x_hbm.at[pl.ds(tbase, SCH)], bufs[0], sins[0])
    for c in range(SNCH):
        b = c & 1
        if c + 1 < SNCH:
            b2 = (c + 1) & 1
            for cp in gout[b2]:
                cp.wait()
            gout[b2] = []
            gin[b2] = pltpu.async_copy(
                x_hbm.at[pl.ds(tbase + (c + 1) * SCH, SCH)], bufs[b2], sins[b2])
        gin[b].wait()
        gout[b] = [
            pltpu.async_copy(bufs[b], out_hbm.at[idx0_v.at[c]], souts[b]),
            pltpu.async_copy(bufs[b], out_hbm.at[idx1_v.at[c]], souts[b]),
        ]
    for b in (0, 1):
        for cp in gout[b]:
            cp.wait()


def _scatter_rows(x, p0w, p1w):
    return pl.kernel(
        _scatter_sc_body,
        out_type=jax.ShapeDtypeStruct((NPAD, H), jnp.float32),
        mesh=_sc_mesh(),
        scratch_types=[
            pltpu.VMEM((SNCH, SCH), jnp.int32),
            pltpu.VMEM((SNCH, SCH), jnp.int32),
            pltpu.VMEM((SCH, H), jnp.float32),
            pltpu.VMEM((SCH, H), jnp.float32),
            pltpu.SemaphoreType.DMA,
            pltpu.SemaphoreType.DMA,
            pltpu.SemaphoreType.DMA,
            pltpu.SemaphoreType.DMA,
        ],
    )(x, p0w, p1w)


def _mlp_body(be_ref, bx_ref, na_ref, xs_ref, wr_ref, wg_ref, wu_ref, wd_ref,
              out_ref):
    b = pl.program_id(0)

    @pl.when(b < na_ref[0])
    def _():
        xb = xs_ref[...]
        h = jnp.dot(xb, wg_ref[0], preferred_element_type=jnp.float32)
        u = jnp.dot(xb, wu_ref[0], preferred_element_type=jnp.float32)
        glu = (h * (1.0 / (1.0 + jnp.exp(-h)))) * u
        glu = glu * wr_ref[:, 0:1]
        out_ref[...] = jnp.dot(glu, wd_ref[0], preferred_element_type=jnp.float32)


def _mlp(xs, wrep, w_gate, w_up, w_down, bexp, bxs, nact):
    grid_spec = pltpu.PrefetchScalarGridSpec(
        num_scalar_prefetch=3,
        grid=(NBLK,),
        in_specs=[
            pl.BlockSpec((TM, H), lambda b, be, bx, na: (bx[b], 0)),
            pl.BlockSpec((TM, 128), lambda b, be, bx, na: (bx[b], 0)),
            pl.BlockSpec((1, H, F), lambda b, be, bx, na: (be[b], 0, 0)),
            pl.BlockSpec((1, H, F), lambda b, be, bx, na: (be[b], 0, 0)),
            pl.BlockSpec((1, F, H), lambda b, be, bx, na: (be[b], 0, 0)),
        ],
        out_specs=pl.BlockSpec((TM, H), lambda b, be, bx, na: (bx[b], 0)),
    )
    return pl.pallas_call(
        _mlp_body,
        grid_spec=grid_spec,
        out_shape=jax.ShapeDtypeStruct((NPAD, H), jnp.float32),
        compiler_params=pltpu.CompilerParams(
            dimension_semantics=("arbitrary",)),
    )(bexp, bxs, nact, xs, wrep, w_gate, w_up, w_down)


CCH = 8
CNCH = C_PER_W // CCH


def _combine_sc_body(rows_hbm, p0_hbm, p1_hbm, out_hbm,
                     idx0_v, idx1_v, a_v0, b_v0, a_v1, b_v1,
                     sg0, sg1, so0, so1):
    wid = lax.axis_index("s") * NC + lax.axis_index("c")
    tbase = wid * C_PER_W
    pltpu.sync_copy(p0_hbm.at[pl.ds(tbase, C_PER_W)], idx0_v)
    pltpu.sync_copy(p1_hbm.at[pl.ds(tbase, C_PER_W)], idx1_v)
    abufs = (a_v0, a_v1)
    bbufs = (b_v0, b_v1)
    sgs = (sg0, sg1)
    sos = (so0, so1)

    def fire(c, s):
        return [
            pltpu.async_copy(rows_hbm.at[idx0_v.at[pl.ds(c * CCH, CCH)]],
                             abufs[s], sgs[s]),
            pltpu.async_copy(rows_hbm.at[idx1_v.at[pl.ds(c * CCH, CCH)]],
                             bbufs[s], sgs[s]),
        ]

    gcp = [[], []]
    ocp = [None, None]
    gcp[0] = fire(0, 0)
    for c in range(CNCH):
        s = c & 1
        if c + 1 < CNCH:
            s2 = (c + 1) & 1
            if ocp[s2] is not None:
                ocp[s2].wait()
                ocp[s2] = None
            gcp[s2] = fire(c + 1, s2)
        for cp in gcp[s]:
            cp.wait()
        a_v, b_v = abufs[s], bbufs[s]

        def row_body(r, _):
            def col_body(jj, _):
                for u in range(8):
                    off = jj * 128 + u * 16
                    a_v[r, pl.ds(off, 16)] = (a_v[r, pl.ds(off, 16)] +
                                              b_v[r, pl.ds(off, 16)])
                return 0
            return lax.fori_loop(0, H // 128, col_body, 0)

        lax.fori_loop(0, CCH, row_body, 0)
        ocp[s] = pltpu.async_copy(
            a_v, out_hbm.at[pl.ds(tbase + c * CCH, CCH)], sos[s])
    for s in (0, 1):
        if ocp[s] is not None:
            ocp[s].wait()


def _combine(yrows, p0, p1):
    return pl.kernel(
        _combine_sc_body,
        out_type=jax.ShapeDtypeStruct((T, H), jnp.float32),
        mesh=_sc_mesh(),
        scratch_types=[
            pltpu.VMEM((C_PER_W,), jnp.int32),
            pltpu.VMEM((C_PER_W,), jnp.int32),
            pltpu.VMEM((CCH, H), jnp.float32),
            pltpu.VMEM((CCH, H), jnp.float32),
            pltpu.VMEM((CCH, H), jnp.float32),
            pltpu.VMEM((CCH, H), jnp.float32),
            pltpu.SemaphoreType.DMA,
            pltpu.SemaphoreType.DMA,
            pltpu.SemaphoreType.DMA,
            pltpu.SemaphoreType.DMA,
        ],
    )(yrows, p0, p1)


def kernel(hidden_states, gate_w, w_gate, w_up, w_down):
    Bb, Ss, Dd = hidden_states.shape
    x = hidden_states.reshape(-1, Dd)
    w2, e2 = _router(x, gate_w)
    row_ids, wrow, bexp, bxs, nact, p01 = _dispatch_plan(e2, w2)
    wrep = jnp.broadcast_to(wrow[:, None], (NPAD, 128))
    p0w = p01[:, 0].reshape(NW, SNCH, SCH)
    p1w = p01[:, 1].reshape(NW, SNCH, SCH)
    xs = _scatter_rows(x, p0w, p1w)
    yrows = _mlp(xs, wrep, w_gate, w_up, w_down, bexp, bxs, nact)
    y = _combine(yrows, p01[:, 0], p01[:, 1])
    return y.reshape(Bb, Ss, Dd)


# speedup vs baseline: 1.9305x; 1.0483x over previous
import functools

import jax
import jax.numpy as jnp
from jax import lax
from jax.experimental import pallas as pl
from jax.experimental.pallas import tpu as pltpu
from jax.experimental.pallas import tpu_sc as plsc

E = 8
K = 2
H = 2048
F = 768
T = 2048
S = T * K
TM = 256
NBLK = S // TM + E
NPAD = NBLK * TM

NC, NS = 2, 16
NW = NC * NS

def _sc_mesh():
    return plsc.VectorSubcoreMesh(core_axis_name="c", subcore_axis_name="s")


def _router_body(x_ref, gw_ref, w_ref, e_ref):
    x = x_ref[...]
    logits = lax.dot_general(x, gw_ref[...], (((1,), (1,)), ((), ())),
                             preferred_element_type=jnp.float32)
    m = jnp.max(logits, axis=1, keepdims=True)
    p = jnp.exp(logits - m)
    sm = p / jnp.sum(p, axis=1, keepdims=True)
    iota = lax.broadcasted_iota(jnp.int32, sm.shape, 1)
    m0 = jnp.max(sm, axis=1, keepdims=True)
    i0 = jnp.min(jnp.where(sm == m0, iota, E), axis=1, keepdims=True)
    masked = jnp.where(iota == i0, -1.0, sm)
    m1 = jnp.max(masked, axis=1, keepdims=True)
    i1 = jnp.min(jnp.where(masked == m1, iota, E), axis=1, keepdims=True)
    ssum = m0 + m1
    w_ref[...] = jnp.concatenate([m0 / ssum, m1 / ssum], axis=1)
    e_ref[...] = jnp.concatenate([i0, i1], axis=1)


def _router(x, gate_w):
    return pl.pallas_call(
        _router_body,
        out_shape=[jax.ShapeDtypeStruct((T, K), jnp.float32),
                   jax.ShapeDtypeStruct((T, K), jnp.int32)],
    )(x, gate_w)


def _dispatch_plan(e2, w2):
    e_flat = e2.reshape(-1)
    w_flat = w2.reshape(-1)
    oh = (e_flat[:, None] == jnp.arange(E, dtype=jnp.int32)[None, :])
    cum = jnp.cumsum(oh.astype(jnp.int32), axis=0)
    rank = jnp.take_along_axis(cum, e_flat[:, None], axis=1)[:, 0] - 1
    counts = cum[-1]
    nblk_e = (counts + TM - 1) // TM
    blk_cum = jnp.cumsum(nblk_e)
    n_active = blk_cum[-1]
    blk_start = blk_cum - nblk_e
    pos = blk_start[e_flat] * TM + rank
    tok = jnp.arange(S, dtype=jnp.int32) // K
    row_ids = jnp.zeros((NPAD,), jnp.int32).at[pos].set(tok)
    wrow = jnp.zeros((NPAD,), jnp.float32).at[pos].set(w_flat)
    bids = jnp.arange(NBLK, dtype=jnp.int32)
    bexp = jnp.searchsorted(blk_cum, bids, side="right").astype(jnp.int32)
    last_e = jnp.searchsorted(blk_cum, n_active - 1, side="right").astype(jnp.int32)
    bexp = jnp.where(bids < n_active, bexp, last_e)
    bxs = jnp.where(bids < n_active, bids, n_active - 1).astype(jnp.int32)
    nact = jnp.reshape(n_active, (1,)).astype(jnp.int32)
    p01 = pos.reshape(T, K).astype(jnp.int32)
    isf = jnp.concatenate([jnp.ones((1,), jnp.int32),
                           (bexp[1:] != bexp[:-1]).astype(jnp.int32)])
    runid = jnp.cumsum(isf) - 1
    slot = (runid & 1).astype(jnp.int32)
    nxtb = jnp.min(jnp.where((isf[None, :] == 1) & (bids[None, :] > bids[:, None]),
                             bids[None, :], NBLK), axis=1)
    hasn = (nxtb < NBLK).astype(jnp.int32)
    nexp = jnp.where(hasn == 1, bexp[jnp.clip(nxtb, 0, NBLK - 1)], 0).astype(jnp.int32)
    return row_ids, wrow, bexp, bxs, nact, p01, isf, slot, nexp, hasn


SCH = 8
C_PER_W = T // NW
SNCH = C_PER_W // SCH


def _scatter_sc_body(x_hbm, p0_hbm, p1_hbm, out_hbm,
                     idx0_v, idx1_v, rows_v0, rows_v1,
                     sin0, sin1, sout0, sout1):
    wid = lax.axis_index("s") * NC + lax.axis_index("c")
    tbase = wid * C_PER_W
    pltpu.sync_copy(p0_hbm.at[wid], idx0_v)
    pltpu.sync_copy(p1_hbm.at[wid], idx1_v)
    bufs = (rows_v0, rows_v1)
    sins = (sin0, sin1)
    souts = (sout0, sout1)
    gin = [None, None]
    gout = [[], []]
    gin[0] = pltpu.async_copy(x_hbm.at[pl.ds(tbase, SCH)], bufs[0], sins[0])
    for c in range(SNCH):
        b = c & 1
        if c + 1 < SNCH:
            b2 = (c + 1) & 1
            for cp in gout[b2]:
                cp.wait()
            gout[b2] = []
            gin[b2] = pltpu.async_copy(
                x_hbm.at[pl.ds(tbase + (c + 1) * SCH, SCH)], bufs[b2], sins[b2])
        gin[b].wait()
        gout[b] = [
            pltpu.async_copy(bufs[b], out_hbm.at[idx0_v.at[c]], souts[b]),
            pltpu.async_copy(bufs[b], out_hbm.at[idx1_v.at[c]], souts[b]),
        ]
    for b in (0, 1):
        for cp in gout[b]:
            cp.wait()


def _scatter_rows(x, p0w, p1w):
    return pl.kernel(
        _scatter_sc_body,
        out_type=jax.ShapeDtypeStruct((NPAD, H), jnp.float32),
        mesh=_sc_mesh(),
        scratch_types=[
            pltpu.VMEM((SNCH, SCH), jnp.int32),
            pltpu.VMEM((SNCH, SCH), jnp.int32),
            pltpu.VMEM((SCH, H), jnp.float32),
            pltpu.VMEM((SCH, H), jnp.float32),
            pltpu.SemaphoreType.DMA,
            pltpu.SemaphoreType.DMA,
            pltpu.SemaphoreType.DMA,
            pltpu.SemaphoreType.DMA,
        ],
    )(x, p0w, p1w)


def _mlp_body(isf_ref, slot_ref, nexp_ref, hasn_ref, be_ref, bx_ref, na_ref,
              xs_ref, wr_ref, wg_hbm, wu_hbm, wd_hbm, out_ref,
              wg_v, wu_v, wd_v, sems):
    b = pl.program_id(0)
    sl = slot_ref[b]

    def _fetch(e, s):
        pltpu.make_async_copy(wg_hbm.at[e], wg_v.at[s], sems.at[s]).start()
        pltpu.make_async_copy(wu_hbm.at[e], wu_v.at[s], sems.at[s]).start()
        pltpu.make_async_copy(wd_hbm.at[e], wd_v.at[s], sems.at[s]).start()

    def _wait(e, s):
        pltpu.make_async_copy(wg_hbm.at[e], wg_v.at[s], sems.at[s]).wait()
        pltpu.make_async_copy(wu_hbm.at[e], wu_v.at[s], sems.at[s]).wait()
        pltpu.make_async_copy(wd_hbm.at[e], wd_v.at[s], sems.at[s]).wait()

    @pl.when(b == 0)
    def _():
        _fetch(be_ref[0], 0)

    @pl.when(isf_ref[b] == 1)
    def _():
        _wait(be_ref[b], sl)

        @pl.when(hasn_ref[b] == 1)
        def _():
            _fetch(nexp_ref[b], 1 - sl)

    @pl.when(b < na_ref[0])
    def _():
        xb = xs_ref[...]
        h = jnp.dot(xb, wg_v[sl], preferred_element_type=jnp.float32)
        u = jnp.dot(xb, wu_v[sl], preferred_element_type=jnp.float32)
        glu = (h * (1.0 / (1.0 + jnp.exp(-h)))) * u
        glu = glu * wr_ref[:, 0:1]
        out_ref[...] = jnp.dot(glu, wd_v[sl], preferred_element_type=jnp.float32)


def _mlp(xs, wrep, w_gate, w_up, w_down, isf, slot, nexp, hasn, bexp, bxs,
         nact):
    grid_spec = pltpu.PrefetchScalarGridSpec(
        num_scalar_prefetch=7,
        grid=(NBLK,),
        in_specs=[
            pl.BlockSpec((TM, H),
                         lambda b, isf, sl, nx, hn, be, bx, na: (bx[b], 0)),
            pl.BlockSpec((TM, 128),
                         lambda b, isf, sl, nx, hn, be, bx, na: (bx[b], 0)),
            pl.BlockSpec(memory_space=pl.ANY),
            pl.BlockSpec(memory_space=pl.ANY),
            pl.BlockSpec(memory_space=pl.ANY),
        ],
        out_specs=pl.BlockSpec((TM, H),
                               lambda b, isf, sl, nx, hn, be, bx, na: (bx[b], 0)),
        scratch_shapes=[
            pltpu.VMEM((2, H, F), jnp.float32),
            pltpu.VMEM((2, H, F), jnp.float32),
            pltpu.VMEM((2, F, H), jnp.float32),
            pltpu.SemaphoreType.DMA((2,)),
        ],
    )
    return pl.pallas_call(
        _mlp_body,
        grid_spec=grid_spec,
        out_shape=jax.ShapeDtypeStruct((NPAD, H), jnp.float32),
        compiler_params=pltpu.CompilerParams(
            dimension_semantics=("arbitrary",)),
    )(isf, slot, nexp, hasn, bexp, bxs, nact, xs, wrep, w_gate, w_up, w_down)


CCH = 8
CNCH = C_PER_W // CCH


def _combine_sc_body(rows_hbm, p0_hbm, p1_hbm, out_hbm,
                     idx0_v, idx1_v, a_v0, b_v0, a_v1, b_v1,
                     sg0, sg1, so0, so1):
    wid = lax.axis_index("s") * NC + lax.axis_index("c")
    tbase = wid * C_PER_W
    pltpu.sync_copy(p0_hbm.at[pl.ds(tbase, C_PER_W)], idx0_v)
    pltpu.sync_copy(p1_hbm.at[pl.ds(tbase, C_PER_W)], idx1_v)
    abufs = (a_v0, a_v1)
    bbufs = (b_v0, b_v1)
    sgs = (sg0, sg1)
    sos = (so0, so1)

    def fire(c, s):
        return [
            pltpu.async_copy(rows_hbm.at[idx0_v.at[pl.ds(c * CCH, CCH)]],
                             abufs[s], sgs[s]),
            pltpu.async_copy(rows_hbm.at[idx1_v.at[pl.ds(c * CCH, CCH)]],
                             bbufs[s], sgs[s]),
        ]

    gcp = [[], []]
    ocp = [None, None]
    gcp[0] = fire(0, 0)
    for c in range(CNCH):
        s = c & 1
        if c + 1 < CNCH:
            s2 = (c + 1) & 1
            if ocp[s2] is not None:
                ocp[s2].wait()
                ocp[s2] = None
            gcp[s2] = fire(c + 1, s2)
        for cp in gcp[s]:
            cp.wait()
        a_v, b_v = abufs[s], bbufs[s]

        def row_body(r, _):
            def col_body(jj, _):
                for u in range(8):
                    off = jj * 128 + u * 16
                    a_v[r, pl.ds(off, 16)] = (a_v[r, pl.ds(off, 16)] +
                                              b_v[r, pl.ds(off, 16)])
                return 0
            return lax.fori_loop(0, H // 128, col_body, 0)

        lax.fori_loop(0, CCH, row_body, 0)
        ocp[s] = pltpu.async_copy(
            a_v, out_hbm.at[pl.ds(tbase + c * CCH, CCH)], sos[s])
    for s in (0, 1):
        if ocp[s] is not None:
            ocp[s].wait()


def _combine(yrows, p0, p1):
    return pl.kernel(
        _combine_sc_body,
        out_type=jax.ShapeDtypeStruct((T, H), jnp.float32),
        mesh=_sc_mesh(),
        scratch_types=[
            pltpu.VMEM((C_PER_W,), jnp.int32),
            pltpu.VMEM((C_PER_W,), jnp.int32),
            pltpu.VMEM((CCH, H), jnp.float32),
            pltpu.VMEM((CCH, H), jnp.float32),
            pltpu.VMEM((CCH, H), jnp.float32),
            pltpu.VMEM((CCH, H), jnp.float32),
            pltpu.SemaphoreType.DMA,
            pltpu.SemaphoreType.DMA,
            pltpu.SemaphoreType.DMA,
            pltpu.SemaphoreType.DMA,
        ],
    )(yrows, p0, p1)


def kernel(hidden_states, gate_w, w_gate, w_up, w_down):
    Bb, Ss, Dd = hidden_states.shape
    x = hidden_states.reshape(-1, Dd)
    w2, e2 = _router(x, gate_w)
    (row_ids, wrow, bexp, bxs, nact, p01,
     isf, slot, nexp, hasn) = _dispatch_plan(e2, w2)
    wrep = jnp.broadcast_to(wrow[:, None], (NPAD, 128))
    p0w = p01[:, 0].reshape(NW, SNCH, SCH)
    p1w = p01[:, 1].reshape(NW, SNCH, SCH)
    xs = _scatter_rows(x, p0w, p1w)
    yrows = _mlp(xs, wrep, w_gate, w_up, w_down, isf, slot, nexp, hasn,
                 bexp, bxs, nact)
    y = _combine(yrows, p01[:, 0], p01[:, 1])
    return y.reshape(Bb, Ss, Dd)


# speedup vs baseline: 2.1957x; 1.1374x over previous
import functools

import jax
import jax.numpy as jnp
from jax import lax
from jax.experimental import pallas as pl
from jax.experimental.pallas import tpu as pltpu
from jax.experimental.pallas import tpu_sc as plsc

E = 8
K = 2
H = 2048
F = 768
T = 2048
S = T * K
TM = 256
NBLK = S // TM + E
NPAD = NBLK * TM

NC, NS = 2, 16
NW = NC * NS

def _sc_mesh():
    return plsc.VectorSubcoreMesh(core_axis_name="c", subcore_axis_name="s")


def _router_body(x_ref, gw_ref, w_ref, pos_ref, bc_ref):
    x = x_ref[...]
    logits = lax.dot_general(x, gw_ref[...], (((1,), (1,)), ((), ())),
                             preferred_element_type=jnp.float32)
    m = jnp.max(logits, axis=1, keepdims=True)
    p = jnp.exp(logits - m)
    sm = p / jnp.sum(p, axis=1, keepdims=True)
    iota = lax.broadcasted_iota(jnp.int32, sm.shape, 1)
    m0 = jnp.max(sm, axis=1, keepdims=True)
    i0 = jnp.min(jnp.where(sm == m0, iota, E), axis=1, keepdims=True)
    masked = jnp.where(iota == i0, -1.0, sm)
    m1 = jnp.max(masked, axis=1, keepdims=True)
    i1 = jnp.min(jnp.where(masked == m1, iota, E), axis=1, keepdims=True)
    ssum = m0 + m1
    w_ref[...] = jnp.concatenate([m0 / ssum, m1 / ssum], axis=1)
    a0 = (iota == i0).astype(jnp.float32)
    a1 = (iota == i1).astype(jnp.float32)
    mm = a0 + a1
    ri = lax.broadcasted_iota(jnp.int32, (T, T), 0)
    ci = lax.broadcasted_iota(jnp.int32, (T, T), 1)
    lst = (ci < ri).astype(jnp.float32)
    cex = jnp.dot(lst, mm, preferred_element_type=jnp.float32)
    counts = jnp.sum(mm, axis=0, keepdims=True)
    nblk = jnp.floor((counts + (TM - 1)) / TM)
    er = lax.broadcasted_iota(jnp.int32, (E, E), 0)
    ec = lax.broadcasted_iota(jnp.int32, (E, E), 1)
    incl = (er <= ec).astype(jnp.float32)
    blk_cum = jnp.dot(nblk, incl, preferred_element_type=jnp.float32)
    off = (blk_cum - nblk) * TM
    pos0 = jnp.sum(a0 * (off + cex), axis=1, keepdims=True)
    pos1 = jnp.sum(a1 * (off + cex), axis=1, keepdims=True)
    pos_ref[...] = jnp.concatenate([pos0, pos1], axis=1).astype(jnp.int32)
    bc_ref[...] = blk_cum.astype(jnp.int32)


def _router(x, gate_w):
    return pl.pallas_call(
        _router_body,
        out_shape=[jax.ShapeDtypeStruct((T, K), jnp.float32),
                   jax.ShapeDtypeStruct((T, K), jnp.int32),
                   jax.ShapeDtypeStruct((1, E), jnp.int32)],
    )(x, gate_w)


def _dispatch_plan(w2, p01, blk_cum2):
    w_flat = w2.reshape(-1)
    pos = p01.reshape(-1)
    blk_cum = blk_cum2[0]
    n_active = blk_cum[-1]
    wrow = jnp.zeros((NPAD,), jnp.float32).at[pos].set(w_flat)
    bids = jnp.arange(NBLK, dtype=jnp.int32)
    bexp = jnp.searchsorted(blk_cum, bids, side="right").astype(jnp.int32)
    last_e = jnp.searchsorted(blk_cum, n_active - 1, side="right").astype(jnp.int32)
    bexp = jnp.where(bids < n_active, bexp, last_e)
    bxs = jnp.where(bids < n_active, bids, n_active - 1).astype(jnp.int32)
    nact = jnp.reshape(n_active, (1,)).astype(jnp.int32)
    isf = jnp.concatenate([jnp.ones((1,), jnp.int32),
                           (bexp[1:] != bexp[:-1]).astype(jnp.int32)])
    runid = jnp.cumsum(isf) - 1
    slot = (runid & 1).astype(jnp.int32)
    nxtb = jnp.min(jnp.where((isf[None, :] == 1) & (bids[None, :] > bids[:, None]),
                             bids[None, :], NBLK), axis=1)
    hasn = (nxtb < NBLK).astype(jnp.int32)
    nexp = jnp.where(hasn == 1, bexp[jnp.clip(nxtb, 0, NBLK - 1)], 0).astype(jnp.int32)
    return wrow, bexp, bxs, nact, isf, slot, nexp, hasn


SCH = 8
C_PER_W = T // NW
SNCH = C_PER_W // SCH


def _scatter_sc_body(x_hbm, p0_hbm, p1_hbm, out_hbm,
                     idx0_v, idx1_v, rows_v0, rows_v1,
                     sin0, sin1, sout0, sout1):
    wid = lax.axis_index("s") * NC + lax.axis_index("c")
    tbase = wid * C_PER_W
    pltpu.sync_copy(p0_hbm.at[wid], idx0_v)
    pltpu.sync_copy(p1_hbm.at[wid], idx1_v)
    bufs = (rows_v0, rows_v1)
    sins = (sin0, sin1)
    souts = (sout0, sout1)
    gin = [None, None]
    gout = [[], []]
    gin[0] = pltpu.async_copy(x_hbm.at[pl.ds(tbase, SCH)], bufs[0], sins[0])
    for c in range(SNCH):
        b = c & 1
        if c + 1 < SNCH:
            b2 = (c + 1) & 1
            for cp in gout[b2]:
                cp.wait()
            gout[b2] = []
            gin[b2] = pltpu.async_copy(
                x_hbm.at[pl.ds(tbase + (c + 1) * SCH, SCH)], bufs[b2], sins[b2])
        gin[b].wait()
        gout[b] = [
            pltpu.async_copy(bufs[b], out_hbm.at[idx0_v.at[c]], souts[b]),
            pltpu.async_copy(bufs[b], out_hbm.at[idx1_v.at[c]], souts[b]),
        ]
    for b in (0, 1):
        for cp in gout[b]:
            cp.wait()


def _scatter_rows(x, p0w, p1w):
    return pl.kernel(
        _scatter_sc_body,
        out_type=jax.ShapeDtypeStruct((NPAD, H), jnp.float32),
        mesh=_sc_mesh(),
        scratch_types=[
            pltpu.VMEM((SNCH, SCH), jnp.int32),
            pltpu.VMEM((SNCH, SCH), jnp.int32),
            pltpu.VMEM((SCH, H), jnp.float32),
            pltpu.VMEM((SCH, H), jnp.float32),
            pltpu.SemaphoreType.DMA,
            pltpu.SemaphoreType.DMA,
            pltpu.SemaphoreType.DMA,
            pltpu.SemaphoreType.DMA,
        ],
    )(x, p0w, p1w)


def _mlp_body(isf_ref, slot_ref, nexp_ref, hasn_ref, be_ref, bx_ref, na_ref,
              xs_ref, wr_ref, wg_hbm, wu_hbm, wd_hbm, out_ref,
              wg_v, wu_v, wd_v, sems):
    b = pl.program_id(0)
    sl = slot_ref[b]

    def _fetch(e, s):
        pltpu.make_async_copy(wg_hbm.at[e], wg_v.at[s], sems.at[s]).start()
        pltpu.make_async_copy(wu_hbm.at[e], wu_v.at[s], sems.at[s]).start()
        pltpu.make_async_copy(wd_hbm.at[e], wd_v.at[s], sems.at[s]).start()

    def _wait(e, s):
        pltpu.make_async_copy(wg_hbm.at[e], wg_v.at[s], sems.at[s]).wait()
        pltpu.make_async_copy(wu_hbm.at[e], wu_v.at[s], sems.at[s]).wait()
        pltpu.make_async_copy(wd_hbm.at[e], wd_v.at[s], sems.at[s]).wait()

    @pl.when(b == 0)
    def _():
        _fetch(be_ref[0], 0)

    @pl.when(isf_ref[b] == 1)
    def _():
        _wait(be_ref[b], sl)

        @pl.when(hasn_ref[b] == 1)
        def _():
            _fetch(nexp_ref[b], 1 - sl)

    @pl.when(b < na_ref[0])
    def _():
        xb = xs_ref[...]
        h = jnp.dot(xb, wg_v[sl], preferred_element_type=jnp.float32)
        u = jnp.dot(xb, wu_v[sl], preferred_element_type=jnp.float32)
        glu = (h * (1.0 / (1.0 + jnp.exp(-h)))) * u
        glu = glu * wr_ref[:, 0:1]
        out_ref[...] = jnp.dot(glu, wd_v[sl], preferred_element_type=jnp.float32)


def _mlp(xs, wrep, w_gate, w_up, w_down, isf, slot, nexp, hasn, bexp, bxs,
         nact):
    grid_spec = pltpu.PrefetchScalarGridSpec(
        num_scalar_prefetch=7,
        grid=(NBLK,),
        in_specs=[
            pl.BlockSpec((TM, H),
                         lambda b, isf, sl, nx, hn, be, bx, na: (bx[b], 0)),
            pl.BlockSpec((TM, 128),
                         lambda b, isf, sl, nx, hn, be, bx, na: (bx[b], 0)),
            pl.BlockSpec(memory_space=pl.ANY),
            pl.BlockSpec(memory_space=pl.ANY),
            pl.BlockSpec(memory_space=pl.ANY),
        ],
        out_specs=pl.BlockSpec((TM, H),
                               lambda b, isf, sl, nx, hn, be, bx, na: (bx[b], 0)),
        scratch_shapes=[
            pltpu.VMEM((2, H, F), jnp.float32),
            pltpu.VMEM((2, H, F), jnp.float32),
            pltpu.VMEM((2, F, H), jnp.float32),
            pltpu.SemaphoreType.DMA((2,)),
        ],
    )
    return pl.pallas_call(
        _mlp_body,
        grid_spec=grid_spec,
        out_shape=jax.ShapeDtypeStruct((NPAD, H), jnp.float32),
        compiler_params=pltpu.CompilerParams(
            dimension_semantics=("arbitrary",)),
    )(isf, slot, nexp, hasn, bexp, bxs, nact, xs, wrep, w_gate, w_up, w_down)


CCH = 8
CNCH = C_PER_W // CCH


def _combine_sc_body(rows_hbm, p0_hbm, p1_hbm, out_hbm,
                     idx0_v, idx1_v, a_v0, b_v0, a_v1, b_v1,
                     sg0, sg1, so0, so1):
    wid = lax.axis_index("s") * NC + lax.axis_index("c")
    tbase = wid * C_PER_W
    pltpu.sync_copy(p0_hbm.at[pl.ds(tbase, C_PER_W)], idx0_v)
    pltpu.sync_copy(p1_hbm.at[pl.ds(tbase, C_PER_W)], idx1_v)
    abufs = (a_v0, a_v1)
    bbufs = (b_v0, b_v1)
    sgs = (sg0, sg1)
    sos = (so0, so1)

    def fire(c, s):
        return [
            pltpu.async_copy(rows_hbm.at[idx0_v.at[pl.ds(c * CCH, CCH)]],
                             abufs[s], sgs[s]),
            pltpu.async_copy(rows_hbm.at[idx1_v.at[pl.ds(c * CCH, CCH)]],
                             bbufs[s], sgs[s]),
        ]

    gcp = [[], []]
    ocp = [None, None]
    gcp[0] = fire(0, 0)
    for c in range(CNCH):
        s = c & 1
        if c + 1 < CNCH:
            s2 = (c + 1) & 1
            if ocp[s2] is not None:
                ocp[s2].wait()
                ocp[s2] = None
            gcp[s2] = fire(c + 1, s2)
        for cp in gcp[s]:
            cp.wait()
        a_v, b_v = abufs[s], bbufs[s]

        def row_body(r, _):
            def col_body(jj, _):
                for u in range(8):
                    off = jj * 128 + u * 16
                    plsc.addupdate(a_v.at[r, pl.ds(off, 16)],
                                   b_v[r, pl.ds(off, 16)])
                return 0
            return lax.fori_loop(0, H // 128, col_body, 0)

        lax.fori_loop(0, CCH, row_body, 0)
        ocp[s] = pltpu.async_copy(
            a_v, out_hbm.at[pl.ds(tbase + c * CCH, CCH)], sos[s])
    for s in (0, 1):
        if ocp[s] is not None:
            ocp[s].wait()


def _combine(yrows, p0, p1):
    return pl.kernel(
        _combine_sc_body,
        out_type=jax.ShapeDtypeStruct((T, H), jnp.float32),
        mesh=_sc_mesh(),
        scratch_types=[
            pltpu.VMEM((C_PER_W,), jnp.int32),
            pltpu.VMEM((C_PER_W,), jnp.int32),
            pltpu.VMEM((CCH, H), jnp.float32),
            pltpu.VMEM((CCH, H), jnp.float32),
            pltpu.VMEM((CCH, H), jnp.float32),
            pltpu.VMEM((CCH, H), jnp.float32),
            pltpu.SemaphoreType.DMA,
            pltpu.SemaphoreType.DMA,
            pltpu.SemaphoreType.DMA,
            pltpu.SemaphoreType.DMA,
        ],
    )(yrows, p0, p1)


def kernel(hidden_states, gate_w, w_gate, w_up, w_down):
    Bb, Ss, Dd = hidden_states.shape
    x = hidden_states.reshape(-1, Dd)
    w2, p01, blk_cum2 = _router(x, gate_w)
    (wrow, bexp, bxs, nact,
     isf, slot, nexp, hasn) = _dispatch_plan(w2, p01, blk_cum2)
    wrep = jnp.broadcast_to(wrow[:, None], (NPAD, 128))
    p0w = p01[:, 0].reshape(NW, SNCH, SCH)
    p1w = p01[:, 1].reshape(NW, SNCH, SCH)
    xs = _scatter_rows(x, p0w, p1w)
    yrows = _mlp(xs, wrep, w_gate, w_up, w_down, isf, slot, nexp, hasn,
                 bexp, bxs, nact)
    y = _combine(yrows, p01[:, 0], p01[:, 1])
    return y.reshape(Bb, Ss, Dd)
